# baseline (speedup 1.0000x reference)
import functools

import jax
import jax.numpy as jnp
from jax import lax
from jax.experimental import pallas as pl
from jax.experimental.pallas import tpu as pltpu
from jax.experimental.pallas import tpu_sc as plsc

N = 10000
E = 320000
F_IN = 128
H = 256
C = 64

NC = 2
NS = 16
NW = NC * NS
K = 112
NP = 10240
EP = 322560
CH = EP // (NW * K)
RPT = NP // NS

RB = 2048
GRID = NP // RB
RBF = 1000

_f32 = jnp.float32


def _mesh():
    return plsc.VectorSubcoreMesh(core_axis_name="c", subcore_axis_name="s")


_SC_PARAMS = pltpu.CompilerParams(use_tc_tiling_on_sc=False)


_DEG_BATCH = 10


def _deg_call(ei, zd):
    @functools.partial(
        pl.kernel,
        out_type=[
            jax.ShapeDtypeStruct((NC, NP), _f32),
            jax.ShapeDtypeStruct((NC, NP), _f32),
        ],
        mesh=_mesh(),
        scratch_types=[
            pltpu.VMEM((CH, 2, K), jnp.int32),
            pltpu.VMEM((K,), _f32),
            pltpu.VMEM_SHARED((NP,), _f32),
            pltpu.VMEM_SHARED((NP,), _f32),
            pltpu.SemaphoreType.DMA,
        ],
        compiler_params=_SC_PARAMS,
    )
    def deg_kernel(ei_hbm, zd_hbm, dout_hbm, din_hbm, iab, ones_v, acc_o,
                   acc_i, sem):
        c = lax.axis_index("c")
        s = lax.axis_index("s")
        w = c * NS + s
        base = s * RPT
        pltpu.sync_copy(ei_hbm.at[w], iab)
        pltpu.sync_copy(zd_hbm, acc_o.at[pl.ds(base, RPT)])
        pltpu.sync_copy(zd_hbm, acc_i.at[pl.ds(base, RPT)])

        @pl.loop(0, K // 16)
        def _(j):
            ones_v[pl.ds(j * 16, 16)] = jnp.full((16,), 1.0, _f32)

        plsc.subcore_barrier()

        @pl.loop(0, CH // _DEG_BATCH)
        def _(i):
            handles = []
            for b in range(_DEG_BATCH):
                j = i * _DEG_BATCH + b
                handles.append(
                    pltpu.async_copy(ones_v, acc_o.at[iab.at[j, 0]], sem, add=True))
                handles.append(
                    pltpu.async_copy(ones_v, acc_i.at[iab.at[j, 1]], sem, add=True))
            for h in handles:
                h.wait()

        plsc.subcore_barrier()
        pltpu.sync_copy(acc_o.at[pl.ds(base, RPT)], dout_hbm.at[c, pl.ds(base, RPT)])
        pltpu.sync_copy(acc_i.at[pl.ds(base, RPT)], din_hbm.at[c, pl.ds(base, RPT)])

    return deg_kernel(ei, zd)


def _prop_call(ei, x, zeros_f, F):
    @functools.partial(
        pl.kernel,
        out_type=jax.ShapeDtypeStruct((NC, NP, F), _f32),
        mesh=_mesh(),
        scratch_types=[
            pltpu.VMEM((3, 2, K), jnp.int32),
            pltpu.VMEM((K, F), _f32),
            pltpu.VMEM((K, F), _f32),
            pltpu.VMEM((K, F), _f32),
            pltpu.SemaphoreType.DMA,
            pltpu.SemaphoreType.DMA,
            pltpu.SemaphoreType.DMA,
            pltpu.SemaphoreType.DMA,
            pltpu.SemaphoreType.DMA,
            pltpu.SemaphoreType.DMA,
            pltpu.SemaphoreType.DMA,
            pltpu.SemaphoreType.DMA,
            pltpu.SemaphoreType.DMA,
            pltpu.VMEM_SHARED((NP, F), _f32),
        ],
        compiler_params=_SC_PARAMS,
    )
    def prop_kernel(ei_hbm, x_hbm, z_hbm, out_hbm, ix3, r0, r1, r2,
                    g0, g1, g2, i0, i1, i2, s0, s1, s2, acc):
        rows = (r0, r1, r2)
        gsem = (g0, g1, g2)
        isem = (i0, i1, i2)
        ssem = (s0, s1, s2)
        c = lax.axis_index("c")
        s = lax.axis_index("s")
        w = c * NS + s
        base = s * RPT

        def wait_idx(b):
            pltpu.make_async_copy(ei_hbm.at[w, 0], ix3.at[b], isem[b]).wait()

        def fire_idx(row, b):
            pltpu.async_copy(ei_hbm.at[w, row], ix3.at[b], isem[b])

        def wait_gather(b):
            pltpu.make_async_copy(x_hbm.at[ix3.at[0, 0]], rows[b], gsem[b]).wait()

        def wait_scatter(b):
            pltpu.make_async_copy(rows[b], acc.at[ix3.at[0, 1]], ssem[b]).wait()

        pltpu.sync_copy(ei_hbm.at[w, 0], ix3.at[0])
        pltpu.sync_copy(z_hbm, acc.at[pl.ds(base, RPT)])
        plsc.subcore_barrier()
        pltpu.async_copy(x_hbm.at[ix3.at[0, 0]], rows[0], gsem[0])
        fire_idx(1, 1)

        @pl.loop(0, CH // 3)
        def _(i):
            for b in range(3):
                j = 3 * i + b
                b1 = (b + 1) % 3
                b2 = (b + 2) % 3
                jn2 = jnp.minimum(j + 2, CH - 1)
                wait_idx(b1)
                pltpu.async_copy(x_hbm.at[ix3.at[b1, 0]], rows[b1], gsem[b1])
                wait_gather(b)
                pltpu.async_copy(rows[b], acc.at[ix3.at[b, 1]], ssem[b], add=True)
                if b == 0:
                    @pl.when(i > 0)
                    def _():
                        wait_scatter(b2)
                else:
                    wait_scatter(b2)
                fire_idx(jn2, b2)

        wait_gather(CH % 3)
        wait_scatter((CH - 1) % 3)
        wait_idx((CH + 1) % 3)
        plsc.subcore_barrier()
        pltpu.sync_copy(acc.at[pl.ds(base, RPT)], out_hbm.at[c, pl.ds(base, RPT)])

    return prop_kernel(ei, x, zeros_f)


def _inv_sqrt_deg(dp):
    d = dp[0] + dp[1]
    return 1.0 / jnp.sqrt(jnp.maximum(d, 1.0))


def _scale_body(x_ref, dop_ref, dip_ref, o_ref, io_ref, ii_ref):
    inv_o = _inv_sqrt_deg(dop_ref[...])
    o_ref[...] = x_ref[...] * inv_o[:, None]
    io_ref[...] = inv_o[:, None]
    ii_ref[...] = _inv_sqrt_deg(dip_ref[...])[:, None]


def _mm_body(p_ref, ii_ref, io_ref, w1_ref, b1_ref, w2_ref, o_ref):
    a = (p_ref[0] + p_ref[1]) * ii_ref[...]
    h = jnp.dot(a, w1_ref[...], preferred_element_type=_f32) + b1_ref[...]
    h = jnp.maximum(h, 0.0)
    t = jnp.dot(h, w2_ref[...], preferred_element_type=_f32)
    o_ref[...] = t * io_ref[...]


def _final_body(p_ref, ii_ref, b2_ref, o_ref):
    o_ref[...] = (p_ref[0] + p_ref[1]) * ii_ref[...] + b2_ref[...]


def kernel(features, edge_index, W1, b1, W2, b2):
    xpad = jnp.pad(features, ((0, NP - N), (0, 0)))
    pad_idx = (N + (jnp.arange(EP - E, dtype=jnp.int32) % 128))
    pad = jnp.stack([pad_idx, pad_idx])
    ei = jnp.concatenate([edge_index.astype(jnp.int32), pad], axis=1)
    ei = ei.reshape(2, NW, CH, K).transpose(1, 2, 0, 3)
    zd = jnp.zeros((RPT,), _f32)
    z128 = jnp.zeros((RPT, F_IN), _f32)
    z64 = jnp.zeros((RPT, C), _f32)

    dout_p, din_p = _deg_call(ei, zd)

    xs, inv_o, inv_i = pl.pallas_call(
        _scale_body,
        grid=(GRID,),
        in_specs=[
            pl.BlockSpec((RB, F_IN), lambda i: (i, 0)),
            pl.BlockSpec((NC, RB), lambda i: (0, i)),
            pl.BlockSpec((NC, RB), lambda i: (0, i)),
        ],
        out_specs=[
            pl.BlockSpec((RB, F_IN), lambda i: (i, 0)),
            pl.BlockSpec((RB, 1), lambda i: (i, 0)),
            pl.BlockSpec((RB, 1), lambda i: (i, 0)),
        ],
        out_shape=[
            jax.ShapeDtypeStruct((NP, F_IN), _f32),
            jax.ShapeDtypeStruct((NP, 1), _f32),
            jax.ShapeDtypeStruct((NP, 1), _f32),
        ],
    )(xpad, dout_p, din_p)

    agg1_p = _prop_call(ei, xs, z128, F_IN)

    t = pl.pallas_call(
        _mm_body,
        grid=(GRID,),
        in_specs=[
            pl.BlockSpec((NC, RB, F_IN), lambda i: (0, i, 0)),
            pl.BlockSpec((RB, 1), lambda i: (i, 0)),
            pl.BlockSpec((RB, 1), lambda i: (i, 0)),
            pl.BlockSpec((F_IN, H), lambda i: (0, 0)),
            pl.BlockSpec((1, H), lambda i: (0, 0)),
            pl.BlockSpec((H, C), lambda i: (0, 0)),
        ],
        out_specs=pl.BlockSpec((RB, C), lambda i: (i, 0)),
        out_shape=jax.ShapeDtypeStruct((NP, C), _f32),
    )(agg1_p, inv_i, inv_o, W1, b1.reshape(1, H), W2)

    agg2_p = _prop_call(ei, t, z64, C)

    out = pl.pallas_call(
        _final_body,
        grid=(N // RBF,),
        in_specs=[
            pl.BlockSpec((NC, RBF, C), lambda i: (0, i, 0)),
            pl.BlockSpec((RBF, 1), lambda i: (i, 0)),
            pl.BlockSpec((1, C), lambda i: (0, 0)),
        ],
        out_specs=pl.BlockSpec((RBF, C), lambda i: (i, 0)),
        out_shape=jax.ShapeDtypeStruct((N, C), _f32),
    )(agg2_p, inv_i, b2.reshape(1, C))

    return out


# speedup vs baseline: 1.0682x; 1.0682x over previous
import functools

import jax
import jax.numpy as jnp
from jax import lax
from jax.experimental import pallas as pl
from jax.experimental.pallas import tpu as pltpu
from jax.experimental.pallas import tpu_sc as plsc

N = 10000
E = 320000
F_IN = 128
H = 256
C = 64

NC = 2
NS = 16
NW = NC * NS
K = 112
NP = 10240
EP = 322560
CH = EP // (NW * K)
RPT = NP // NS

RB = 2048
GRID = NP // RB
RBF = 1000

_f32 = jnp.float32


def _mesh():
    return plsc.VectorSubcoreMesh(core_axis_name="c", subcore_axis_name="s")


_SC_PARAMS = pltpu.CompilerParams(use_tc_tiling_on_sc=False)


_DEG_BATCH = 10


def _deg_call(ei, zd):
    @functools.partial(
        pl.kernel,
        out_type=[
            jax.ShapeDtypeStruct((NC, NP), _f32),
            jax.ShapeDtypeStruct((NC, NP), _f32),
        ],
        mesh=_mesh(),
        scratch_types=[
            pltpu.VMEM((CH, K), jnp.int32),
            pltpu.VMEM((CH, K), jnp.int32),
            pltpu.VMEM((K,), _f32),
            pltpu.VMEM_SHARED((NP,), _f32),
            pltpu.VMEM_SHARED((NP,), _f32),
            pltpu.SemaphoreType.DMA,
        ],
        compiler_params=_SC_PARAMS,
    )
    def deg_kernel(ei_hbm, zd_hbm, dout_hbm, din_hbm, isb, idb, ones_v, acc_o,
                   acc_i, sem):
        c = lax.axis_index("c")
        s = lax.axis_index("s")
        w = c * NS + s
        base = s * RPT
        pltpu.sync_copy(ei_hbm.at[0, w], isb)
        pltpu.sync_copy(ei_hbm.at[1, w], idb)
        pltpu.sync_copy(zd_hbm, acc_o.at[pl.ds(base, RPT)])
        pltpu.sync_copy(zd_hbm, acc_i.at[pl.ds(base, RPT)])

        @pl.loop(0, K // 16)
        def _(j):
            ones_v[pl.ds(j * 16, 16)] = jnp.full((16,), 1.0, _f32)

        plsc.subcore_barrier()

        @pl.loop(0, CH // _DEG_BATCH)
        def _(i):
            handles = []
            for b in range(_DEG_BATCH):
                j = i * _DEG_BATCH + b
                handles.append(
                    pltpu.async_copy(ones_v, acc_o.at[isb.at[j]], sem, add=True))
                handles.append(
                    pltpu.async_copy(ones_v, acc_i.at[idb.at[j]], sem, add=True))
            for h in handles:
                h.wait()

        plsc.subcore_barrier()
        pltpu.sync_copy(acc_o.at[pl.ds(base, RPT)], dout_hbm.at[c, pl.ds(base, RPT)])
        pltpu.sync_copy(acc_i.at[pl.ds(base, RPT)], din_hbm.at[c, pl.ds(base, RPT)])

    return deg_kernel(ei, zd)


def _prop_call(ei, x, zeros_f, F):
    @functools.partial(
        pl.kernel,
        out_type=jax.ShapeDtypeStruct((NC, NP, F), _f32),
        mesh=_mesh(),
        scratch_types=[
            pltpu.VMEM((3, K), jnp.int32),
            pltpu.VMEM((3, K), jnp.int32),
            pltpu.VMEM((K, F), _f32),
            pltpu.VMEM((K, F), _f32),
            pltpu.VMEM((K, F), _f32),
            pltpu.SemaphoreType.DMA,
            pltpu.SemaphoreType.DMA,
            pltpu.SemaphoreType.DMA,
            pltpu.SemaphoreType.DMA,
            pltpu.SemaphoreType.DMA,
            pltpu.SemaphoreType.DMA,
            pltpu.SemaphoreType.DMA,
            pltpu.SemaphoreType.DMA,
            pltpu.SemaphoreType.DMA,
            pltpu.VMEM_SHARED((NP, F), _f32),
        ],
        compiler_params=_SC_PARAMS,
    )
    def prop_kernel(ei_hbm, x_hbm, z_hbm, out_hbm, is3, id3, r0, r1, r2,
                    g0, g1, g2, i0, i1, i2, s0, s1, s2, acc):
        rows = (r0, r1, r2)
        gsem = (g0, g1, g2)
        isem = (i0, i1, i2)
        ssem = (s0, s1, s2)
        c = lax.axis_index("c")
        s = lax.axis_index("s")
        w = c * NS + s
        base = s * RPT

        def wait_idx(b):
            pltpu.make_async_copy(ei_hbm.at[0, w, 0], is3.at[b], isem[b]).wait()
            pltpu.make_async_copy(ei_hbm.at[1, w, 0], id3.at[b], isem[b]).wait()

        def fire_idx(row, b):
            pltpu.async_copy(ei_hbm.at[0, w, row], is3.at[b], isem[b])
            pltpu.async_copy(ei_hbm.at[1, w, row], id3.at[b], isem[b])

        def wait_gather(b):
            pltpu.make_async_copy(x_hbm.at[is3.at[0]], rows[b], gsem[b]).wait()

        def wait_scatter(b):
            pltpu.make_async_copy(rows[b], acc.at[id3.at[0]], ssem[b]).wait()

        pltpu.sync_copy(ei_hbm.at[0, w, 0], is3.at[0])
        pltpu.sync_copy(ei_hbm.at[1, w, 0], id3.at[0])
        pltpu.sync_copy(z_hbm, acc.at[pl.ds(base, RPT)])
        plsc.subcore_barrier()
        pltpu.async_copy(x_hbm.at[is3.at[0]], rows[0], gsem[0])
        fire_idx(1, 1)

        @pl.loop(0, CH // 3)
        def _(i):
            for b in range(3):
                j = 3 * i + b
                b1 = (b + 1) % 3
                b2 = (b + 2) % 3
                jn2 = jnp.minimum(j + 2, CH - 1)
                wait_idx(b1)
                if b == 0:
                    @pl.when(i > 0)
                    def _():
                        wait_scatter(b2)
                else:
                    wait_scatter(b2)
                pltpu.async_copy(x_hbm.at[is3.at[b1]], rows[b1], gsem[b1])
                wait_gather(b)
                pltpu.async_copy(rows[b], acc.at[id3.at[b]], ssem[b], add=True)
                fire_idx(jn2, b2)

        wait_gather(CH % 3)
        wait_scatter((CH - 1) % 3)
        wait_idx((CH + 1) % 3)
        plsc.subcore_barrier()
        pltpu.sync_copy(acc.at[pl.ds(base, RPT)], out_hbm.at[c, pl.ds(base, RPT)])

    return prop_kernel(ei, x, zeros_f)


def _inv_sqrt_deg(dp):
    d = dp[0] + dp[1]
    return 1.0 / jnp.sqrt(jnp.maximum(d, 1.0))


def _scale_body(x_ref, dop_ref, dip_ref, o_ref, io_ref, ii_ref):
    inv_o = _inv_sqrt_deg(dop_ref[...])
    o_ref[...] = x_ref[...] * inv_o[:, None]
    io_ref[...] = inv_o[:, None]
    ii_ref[...] = _inv_sqrt_deg(dip_ref[...])[:, None]


def _mm_body(p_ref, ii_ref, io_ref, w1_ref, b1_ref, w2_ref, o_ref):
    a = (p_ref[0] + p_ref[1]) * ii_ref[...]
    h = jnp.dot(a, w1_ref[...], preferred_element_type=_f32) + b1_ref[...]
    h = jnp.maximum(h, 0.0)
    t = jnp.dot(h, w2_ref[...], preferred_element_type=_f32)
    o_ref[...] = t * io_ref[...]


def _final_body(p_ref, ii_ref, b2_ref, o_ref):
    o_ref[...] = (p_ref[0] + p_ref[1]) * ii_ref[...] + b2_ref[...]


def kernel(features, edge_index, W1, b1, W2, b2):
    xpad = jnp.pad(features, ((0, NP - N), (0, 0)))
    pad_idx = (N + (jnp.arange(EP - E, dtype=jnp.int32) % 128))
    pad = jnp.stack([pad_idx, pad_idx])
    ei = jnp.concatenate([edge_index.astype(jnp.int32), pad], axis=1)
    ei = ei.reshape(2, NW, CH, K)
    zd = jnp.zeros((RPT,), _f32)
    z128 = jnp.zeros((RPT, F_IN), _f32)
    z64 = jnp.zeros((RPT, C), _f32)

    dout_p, din_p = _deg_call(ei, zd)

    xs, inv_o, inv_i = pl.pallas_call(
        _scale_body,
        grid=(GRID,),
        in_specs=[
            pl.BlockSpec((RB, F_IN), lambda i: (i, 0)),
            pl.BlockSpec((NC, RB), lambda i: (0, i)),
            pl.BlockSpec((NC, RB), lambda i: (0, i)),
        ],
        out_specs=[
            pl.BlockSpec((RB, F_IN), lambda i: (i, 0)),
            pl.BlockSpec((RB, 1), lambda i: (i, 0)),
            pl.BlockSpec((RB, 1), lambda i: (i, 0)),
        ],
        out_shape=[
            jax.ShapeDtypeStruct((NP, F_IN), _f32),
            jax.ShapeDtypeStruct((NP, 1), _f32),
            jax.ShapeDtypeStruct((NP, 1), _f32),
        ],
    )(xpad, dout_p, din_p)

    agg1_p = _prop_call(ei, xs, z128, F_IN)

    t = pl.pallas_call(
        _mm_body,
        grid=(GRID,),
        in_specs=[
            pl.BlockSpec((NC, RB, F_IN), lambda i: (0, i, 0)),
            pl.BlockSpec((RB, 1), lambda i: (i, 0)),
            pl.BlockSpec((RB, 1), lambda i: (i, 0)),
            pl.BlockSpec((F_IN, H), lambda i: (0, 0)),
            pl.BlockSpec((1, H), lambda i: (0, 0)),
            pl.BlockSpec((H, C), lambda i: (0, 0)),
        ],
        out_specs=pl.BlockSpec((RB, C), lambda i: (i, 0)),
        out_shape=jax.ShapeDtypeStruct((NP, C), _f32),
    )(agg1_p, inv_i, inv_o, W1, b1.reshape(1, H), W2)

    agg2_p = _prop_call(ei, t, z64, C)

    out = pl.pallas_call(
        _final_body,
        grid=(N // RBF,),
        in_specs=[
            pl.BlockSpec((NC, RBF, C), lambda i: (0, i, 0)),
            pl.BlockSpec((RBF, 1), lambda i: (i, 0)),
            pl.BlockSpec((1, C), lambda i: (0, 0)),
        ],
        out_specs=pl.BlockSpec((RBF, C), lambda i: (i, 0)),
        out_shape=jax.ShapeDtypeStruct((N, C), _f32),
    )(agg2_p, inv_i, b2.reshape(1, C))

    return out
